# baseline (speedup 1.0000x reference)
import functools
import math

import jax
import jax.numpy as jnp
import numpy as np
from jax.experimental import pallas as pl
from jax.experimental.pallas import tpu as pltpu

_B, _Q, _DM = 2, 5440, 256
_G, _L, _K = 8, 4, 4
_DH = _DM // _G
_SPATIAL = np.array([[64, 64], [32, 32], [16, 16], [8, 8]], dtype=np.int64)
_LEVEL_START = np.array([0, 4096, 5120, 5376], dtype=np.int64)
_LEN_IN = 5440


def _matmul_bias_kernel(x_ref, w_ref, b_ref, o_ref):
    o_ref[...] = (
        jnp.dot(x_ref[...], w_ref[...], preferred_element_type=jnp.float32)
        + b_ref[...]
    )


def _matmul_bias(x, w, b, bm=2176):
    m, k = x.shape
    _, n = w.shape
    grid = (m // bm,)
    return pl.pallas_call(
        _matmul_bias_kernel,
        grid=grid,
        in_specs=[
            pl.BlockSpec((bm, k), lambda i: (i, 0)),
            pl.BlockSpec((k, n), lambda i: (0, 0)),
            pl.BlockSpec((1, n), lambda i: (0, 0)),
        ],
        out_specs=pl.BlockSpec((bm, n), lambda i: (i, 0)),
        out_shape=jax.ShapeDtypeStruct((m, n), jnp.float32),
    )(x, w, b.reshape(1, n))


def _bilinear_sample(value, h_im, w_im):
    Bv, H, W, Gv, Dv = value.shape
    _, Qv, _, Kv = h_im.shape
    h0 = jnp.floor(h_im)
    w0 = jnp.floor(w_im)
    lh = h_im - h0
    lw = w_im - w0
    hh = 1.0 - lh
    hw = 1.0 - lw
    h0i = h0.astype(jnp.int32)
    w0i = w0.astype(jnp.int32)
    h1i = h0i + 1
    w1i = w0i + 1
    value_flat = value.transpose(0, 3, 1, 2, 4).reshape(Bv * Gv, H * W, Dv)

    def gather(h_idx, w_idx):
        mask = (h_idx >= 0) & (h_idx < H) & (w_idx >= 0) & (w_idx < W)
        hc = jnp.clip(h_idx, 0, H - 1)
        wc = jnp.clip(w_idx, 0, W - 1)
        idx = hc * W + wc
        idx_flat = idx.transpose(0, 2, 1, 3).reshape(Bv * Gv, Qv * Kv)
        idx_b = jnp.broadcast_to(idx_flat[:, :, None], (Bv * Gv, Qv * Kv, Dv))
        g = jnp.take_along_axis(value_flat, idx_b, axis=1)
        g = g.reshape(Bv, Gv, Qv, Kv, Dv).transpose(0, 2, 1, 3, 4)
        return g * mask.astype(value.dtype)[..., None]

    v1 = gather(h0i, w0i)
    v2 = gather(h0i, w1i)
    v3 = gather(h1i, w0i)
    v4 = gather(h1i, w1i)
    out = (v1 * (hh * hw)[..., None] + v2 * (hh * lw)[..., None]
           + v3 * (lh * hw)[..., None] + v4 * (lh * lw)[..., None])
    valid = (h_im > -1) & (w_im > -1) & (h_im < H) & (w_im < W)
    return out * valid.astype(value.dtype)[..., None]


def kernel(query, reference_points, input_flatten, spatial_shapes,
           level_start_index, W_samp, b_samp, W_attn, b_attn, W_val, b_val,
           W_out, b_out):
    q2 = query.reshape(_B * _Q, _DM)
    value = _matmul_bias(input_flatten.reshape(_B * _LEN_IN, _DM), W_val, b_val)
    value = value.reshape(_B, _LEN_IN, _G, _DH)
    samp_off = _matmul_bias(q2, W_samp, b_samp).reshape(_B, _Q, _G, _L, _K, 2)
    attn_logits = _matmul_bias(q2, W_attn, b_attn).reshape(_B, _Q, _G, _L * _K)

    normalizer = jnp.asarray(
        np.stack([_SPATIAL[:, 1], _SPATIAL[:, 0]], -1), dtype=jnp.float32)
    sampling_locations = (reference_points[:, :, None, :, None, :]
                          + samp_off / normalizer[None, None, None, :, None, :])
    attn_logits = attn_logits - jnp.max(attn_logits, axis=-1, keepdims=True)
    attn = jnp.exp(attn_logits)
    attn = attn / jnp.sum(attn, axis=-1, keepdims=True)
    attn = attn.reshape(_B, _Q, _G, _L, _K)
    output = jnp.zeros((_B, _Q, _G, _DH), dtype=jnp.float32)
    for level in range(_L):
        Hh = int(_SPATIAL[level, 0])
        Ww = int(_SPATIAL[level, 1])
        start = int(_LEVEL_START[level])
        value_level = value[:, start:start + Hh * Ww].reshape(_B, Hh, Ww, _G, _DH)
        loc = sampling_locations[:, :, :, level]
        h_im = loc[..., 1] * Hh - 0.5
        w_im = loc[..., 0] * Ww - 0.5
        sampled = _bilinear_sample(value_level, h_im, w_im)
        output = output + (sampled * attn[:, :, :, level][..., None]).sum(axis=3)
    out = _matmul_bias(output.reshape(_B * _Q, _DM), W_out, b_out)
    return out.reshape(_B, _Q, _DM)


# speedup vs baseline: 1146.0988x; 1146.0988x over previous
import functools
import math

import jax
import jax.numpy as jnp
import numpy as np
from jax import lax
from jax.experimental import pallas as pl
from jax.experimental.pallas import tpu as pltpu
from jax.experimental.pallas import tpu_sc as plsc

_B, _Q, _DM = 2, 5440, 256
_G, _L, _K = 8, 4, 4
_DH = _DM // _G
_SPATIAL = np.array([[64, 64], [32, 32], [16, 16], [8, 8]], dtype=np.int64)
_LEVEL_START = np.array([0, 4096, 5120, 5376], dtype=np.int64)
_LEN_IN = 5440
_N_OUT = _B * _Q * _G
_QB = 544
_NTILES = 32
_CHUNK = _N_OUT // _NTILES
_NB = 8
_NROW = _NB * _L * _K

_lane = np.arange(_G * _L * _K)
_lane_l = (_lane // _K) % _L
_LANE_H = _SPATIAL[_lane_l, 0].astype(np.float32).reshape(1, -1)
_LANE_W = _SPATIAL[_lane_l, 1].astype(np.float32).reshape(1, -1)
_LANE_HI = _SPATIAL[_lane_l, 0].astype(np.int32).reshape(1, -1)
_LANE_WI = _SPATIAL[_lane_l, 1].astype(np.int32).reshape(1, -1)
_LANE_START = _LEVEL_START[_lane_l].astype(np.int32).reshape(1, -1)
_LANE_G = (_lane // (_L * _K)).astype(np.int32).reshape(1, -1)
_LSEL = np.zeros((_L, _G * _L * _K), np.float32)
_LSEL[_lane_l, _lane] = 1.0


def _matmul_bias_kernel(x_ref, w_ref, b_ref, o_ref):
    o_ref[...] = (
        jnp.dot(x_ref[...], w_ref[...], preferred_element_type=jnp.float32)
        + b_ref[...]
    )


def _matmul_bias(x, w, b, bm=2176):
    m, k = x.shape
    _, n = w.shape
    return pl.pallas_call(
        _matmul_bias_kernel,
        grid=(m // bm,),
        in_specs=[
            pl.BlockSpec((bm, k), lambda i: (i, 0)),
            pl.BlockSpec((k, n), lambda i: (0, 0)),
            pl.BlockSpec((1, n), lambda i: (0, 0)),
        ],
        out_specs=pl.BlockSpec((bm, n), lambda i: (i, 0)),
        out_shape=jax.ShapeDtypeStruct((m, n), jnp.float32),
    )(x, w, b.reshape(1, n))


def _prep_kernel(q_ref, wx_ref, wy_ref, wa_ref, bx_ref, by_ref, ba_ref,
                 rx_ref, ry_ref, lsel_ref, hi_ref, wi_ref, start_ref, g_ref,
                 i0_ref, i1_ref, i2_ref, i3_ref,
                 w0_ref, w1_ref, w2_ref, w3_ref, lg_ref):
    qb = q_ref[...]
    sx = jnp.dot(qb, wx_ref[...], preferred_element_type=jnp.float32) + bx_ref[...]
    sy = jnp.dot(qb, wy_ref[...], preferred_element_type=jnp.float32) + by_ref[...]
    lg_ref[...] = (
        jnp.dot(qb, wa_ref[...], preferred_element_type=jnp.float32) + ba_ref[...]
    )
    lsel = lsel_ref[...]
    rx = jnp.dot(rx_ref[...], lsel, preferred_element_type=jnp.float32,
                 precision=lax.Precision.HIGHEST)
    ry = jnp.dot(ry_ref[...], lsel, preferred_element_type=jnp.float32,
                 precision=lax.Precision.HIGHEST)
    hf = hi_ref[...].astype(jnp.float32)
    wf = wi_ref[...].astype(jnp.float32)
    lx = rx + sx * (1.0 / wf)
    ly = ry + sy * (1.0 / hf)
    wim = lx * wf - 0.5
    him = ly * hf - 0.5
    h0f = jnp.floor(him)
    w0f = jnp.floor(wim)
    lh = him - h0f
    lw = wim - w0f
    hh = 1.0 - lh
    hw = 1.0 - lw
    h0 = h0f.astype(jnp.int32)
    w0 = w0f.astype(jnp.int32)
    h1 = h0 + 1
    w1 = w0 + 1
    validf = ((him > -1.0) & (wim > -1.0) & (him < hf) & (wim < wf)).astype(
        jnp.float32)
    hi = hi_ref[...]
    wi = wi_ref[...]
    start = start_ref[...]
    glane = g_ref[...]
    boff = pl.program_id(0) * (_LEN_IN * _G)
    iouts = (i0_ref, i1_ref, i2_ref, i3_ref)
    wouts = (w0_ref, w1_ref, w2_ref, w3_ref)
    corners = ((h0, w0, hh * hw), (h0, w1, hh * lw),
               (h1, w0, lh * hw), (h1, w1, lh * lw))
    for (hc_, wc_, bw), i_ref, w_ref in zip(corners, iouts, wouts):
        m = ((hc_ >= 0) & (hc_ < hi) & (wc_ >= 0) & (wc_ < wi)).astype(
            jnp.float32)
        hcl = jnp.clip(hc_, 0, hi - 1)
        wcl = jnp.clip(wc_, 0, wi - 1)
        i_ref[...] = (start + hcl * wi + wcl) * _G + glane + boff
        w_ref[...] = bw * m * validf


def _prep(q2, wx, wy, wa, bx, by, ba, rx, ry):
    nlane = _G * _L * _K
    nb = _Q // _QB
    io = jax.ShapeDtypeStruct((_B * _Q, nlane), jnp.int32)
    wo = jax.ShapeDtypeStruct((_B * _Q, nlane), jnp.float32)
    blk = lambda i, b=None: (i, 0)
    return pl.pallas_call(
        _prep_kernel,
        grid=(_B, nb),
        in_specs=[
            pl.BlockSpec((_QB, _DM), lambda b, i: (b * nb + i, 0)),
            pl.BlockSpec((_DM, nlane), lambda b, i: (0, 0)),
            pl.BlockSpec((_DM, nlane), lambda b, i: (0, 0)),
            pl.BlockSpec((_DM, nlane), lambda b, i: (0, 0)),
            pl.BlockSpec((1, nlane), lambda b, i: (0, 0)),
            pl.BlockSpec((1, nlane), lambda b, i: (0, 0)),
            pl.BlockSpec((1, nlane), lambda b, i: (0, 0)),
            pl.BlockSpec((_QB, _L), lambda b, i: (b * nb + i, 0)),
            pl.BlockSpec((_QB, _L), lambda b, i: (b * nb + i, 0)),
            pl.BlockSpec((_L, nlane), lambda b, i: (0, 0)),
            pl.BlockSpec((1, nlane), lambda b, i: (0, 0)),
            pl.BlockSpec((1, nlane), lambda b, i: (0, 0)),
            pl.BlockSpec((1, nlane), lambda b, i: (0, 0)),
            pl.BlockSpec((1, nlane), lambda b, i: (0, 0)),
        ],
        out_specs=[pl.BlockSpec((_QB, nlane), lambda b, i: (b * nb + i, 0))] * 9,
        out_shape=[io, io, io, io, wo, wo, wo, wo, wo],
    )(q2, wx, wy, wa, bx, by, ba, rx, ry,
      jnp.asarray(_LSEL), jnp.asarray(_LANE_HI), jnp.asarray(_LANE_WI),
      jnp.asarray(_LANE_START), jnp.asarray(_LANE_G))


def _lane_bcast(x, idx):
    dn = lax.GatherDimensionNumbers(
        offset_dims=(), collapsed_slice_dims=(0,), start_index_map=(0,))
    return lax.gather(x, idx[:, None], dn, slice_sizes=(1,),
                      mode=lax.GatherScatterMode.PROMISE_IN_BOUNDS)


def _all_max(x):
    lane = lax.iota(jnp.int32, 16)
    for k in range(4):
        x = jnp.maximum(x, _lane_bcast(x, lane ^ (1 << k)))
    return x


def _all_sum(x):
    lane = lax.iota(jnp.int32, 16)
    for k in range(4):
        x = x + _lane_bcast(x, lane ^ (1 << k))
    return x


def _sc_body(table, i0, i1, i2, i3, w0, w1, w2, w3, lg, out,
             i0v, i1v, i2v, i3v, r0v, r1v, r2v, r3v,
             w0v, w1v, w2v, w3v, lgv_s, outv, sem):
    wid = lax.axis_index("s") * 2 + lax.axis_index("c")
    base = wid * _CHUNK
    ivs = (i0v, i1v, i2v, i3v)
    rvs = (r0v, r1v, r2v, r3v)
    wvs = (w0v, w1v, w2v, w3v)
    ihs = (i0, i1, i2, i3)
    whs = (w0, w1, w2, w3)

    def block(bi, carry):
        bo = base + bi * _NB
        for ih, iv in zip(ihs, ivs):
            pltpu.sync_copy(ih.at[pl.ds(bo * 16, _NROW)], iv)
        for wh, wv in zip(whs, wvs):
            pltpu.sync_copy(wh.at[pl.ds(bo, _NB)], wv)
        pltpu.sync_copy(lg.at[pl.ds(bo, _NB)], lgv_s)
        descs = [pltpu.async_copy(table.at[iv], rv, sem)
                 for iv, rv in zip(ivs, rvs)]
        for d in descs:
            d.wait()
        for o in range(_NB):
            lgv = lgv_s[o]
            mx = _all_max(lgv)
            e = jnp.exp(lgv - mx)
            rinv = 1.0 / _all_sum(e)
            ew = e * rinv
            wfin = [wvs[c][o] * ew for c in range(4)]
            acc0 = jnp.zeros((16,), jnp.float32)
            acc1 = jnp.zeros((16,), jnp.float32)
            for c in range(4):
                rv = rvs[c]
                for j in range(16):
                    wb = _lane_bcast(wfin[c], jnp.full((16,), j, jnp.int32))
                    row = o * 16 + j
                    acc0 = acc0 + wb * rv[row, pl.ds(0, 16)]
                    acc1 = acc1 + wb * rv[row, pl.ds(16, 16)]
            outv[o, pl.ds(0, 16)] = acc0
            outv[o, pl.ds(16, 16)] = acc1
        pltpu.sync_copy(outv, out.at[pl.ds(bo, _NB)])
        return carry

    lax.fori_loop(0, _CHUNK // _NB, block, 0)


@functools.partial(jax.jit)
def _sc_sample(table, i0, i1, i2, i3, w0, w1, w2, w3, lg):
    mesh = plsc.VectorSubcoreMesh(core_axis_name="c", subcore_axis_name="s",
                                  num_cores=2, num_subcores=16)
    f = pl.kernel(
        _sc_body,
        out_type=jax.ShapeDtypeStruct((_N_OUT, _DH), jnp.float32),
        mesh=mesh,
        scratch_types=[
            pltpu.VMEM((_NROW,), jnp.int32),
            pltpu.VMEM((_NROW,), jnp.int32),
            pltpu.VMEM((_NROW,), jnp.int32),
            pltpu.VMEM((_NROW,), jnp.int32),
            pltpu.VMEM((_NROW, _DH), jnp.float32),
            pltpu.VMEM((_NROW, _DH), jnp.float32),
            pltpu.VMEM((_NROW, _DH), jnp.float32),
            pltpu.VMEM((_NROW, _DH), jnp.float32),
            pltpu.VMEM((_NB, 16), jnp.float32),
            pltpu.VMEM((_NB, 16), jnp.float32),
            pltpu.VMEM((_NB, 16), jnp.float32),
            pltpu.VMEM((_NB, 16), jnp.float32),
            pltpu.VMEM((_NB, 16), jnp.float32),
            pltpu.VMEM((_NB, _DH), jnp.float32),
            pltpu.SemaphoreType.DMA,
        ],
        compiler_params=pltpu.CompilerParams(use_tc_tiling_on_sc=False),
    )
    return f(table, i0, i1, i2, i3, w0, w1, w2, w3, lg)


def kernel(query, reference_points, input_flatten, spatial_shapes,
           level_start_index, W_samp, b_samp, W_attn, b_attn, W_val, b_val,
           W_out, b_out):
    q2 = query.reshape(_B * _Q, _DM)
    ws = W_samp.reshape(_DM, _G, _L, _K, 2)
    wx = ws[..., 0].reshape(_DM, -1)
    wy = ws[..., 1].reshape(_DM, -1)
    bs = b_samp.reshape(_G, _L, _K, 2)
    bx = bs[..., 0].reshape(1, -1)
    by = bs[..., 1].reshape(1, -1)
    rx = reference_points[..., 0].reshape(_B * _Q, _L)
    ry = reference_points[..., 1].reshape(_B * _Q, _L)

    i0, i1, i2, i3, w0, w1, w2, w3, lgq = _prep(
        q2, wx, wy, W_attn, bx, by, b_attn.reshape(1, -1), rx, ry)

    value = _matmul_bias(input_flatten.reshape(_B * _LEN_IN, _DM), W_val, b_val)
    table = value.reshape(_B * _LEN_IN * _G, _DH)

    flat_i = lambda a: a.reshape(_N_OUT * 16)
    flat_w = lambda a: a.reshape(_N_OUT, 16)
    out_sc = _sc_sample(table,
                        flat_i(i0), flat_i(i1), flat_i(i2), flat_i(i3),
                        flat_w(w0), flat_w(w1), flat_w(w2), flat_w(w3),
                        flat_w(lgq))

    out = _matmul_bias(out_sc.reshape(_B * _Q, _DM), W_out, b_out)
    return out.reshape(_B, _Q, _DM)


# speedup vs baseline: 1639.9641x; 1.4309x over previous
import functools
import math

import jax
import jax.numpy as jnp
import numpy as np
from jax import lax
from jax.experimental import pallas as pl
from jax.experimental.pallas import tpu as pltpu
from jax.experimental.pallas import tpu_sc as plsc

_B, _Q, _DM = 2, 5440, 256
_G, _L, _K = 8, 4, 4
_DH = _DM // _G
_SPATIAL = np.array([[64, 64], [32, 32], [16, 16], [8, 8]], dtype=np.int64)
_LEVEL_START = np.array([0, 4096, 5120, 5376], dtype=np.int64)
_LEN_IN = 5440
_N_OUT = _B * _Q * _G
_QB = 544
_NTILES = 32
_CHUNK = _N_OUT // _NTILES
_NB = 8
_NROW = _NB * _L * _K

_lane = np.arange(_G * _L * _K)
_lane_l = (_lane // _K) % _L
_LANE_H = _SPATIAL[_lane_l, 0].astype(np.float32).reshape(1, -1)
_LANE_W = _SPATIAL[_lane_l, 1].astype(np.float32).reshape(1, -1)
_LANE_HI = _SPATIAL[_lane_l, 0].astype(np.int32).reshape(1, -1)
_LANE_WI = _SPATIAL[_lane_l, 1].astype(np.int32).reshape(1, -1)
_LANE_START = _LEVEL_START[_lane_l].astype(np.int32).reshape(1, -1)
_LANE_G = (_lane // (_L * _K)).astype(np.int32).reshape(1, -1)
_LSEL = np.zeros((_L, _G * _L * _K), np.float32)
_LSEL[_lane_l, _lane] = 1.0


def _matmul_bias_kernel(x_ref, w_ref, b_ref, o_ref):
    o_ref[...] = (
        jnp.dot(x_ref[...], w_ref[...], preferred_element_type=jnp.float32)
        + b_ref[...]
    )


def _matmul_bias(x, w, b, bm=2176):
    m, k = x.shape
    _, n = w.shape
    return pl.pallas_call(
        _matmul_bias_kernel,
        grid=(m // bm,),
        in_specs=[
            pl.BlockSpec((bm, k), lambda i: (i, 0)),
            pl.BlockSpec((k, n), lambda i: (0, 0)),
            pl.BlockSpec((1, n), lambda i: (0, 0)),
        ],
        out_specs=pl.BlockSpec((bm, n), lambda i: (i, 0)),
        out_shape=jax.ShapeDtypeStruct((m, n), jnp.float32),
    )(x, w, b.reshape(1, n))


def _prep_kernel(q_ref, wx_ref, wy_ref, wa_ref, bx_ref, by_ref, ba_ref,
                 rx_ref, ry_ref, lsel_ref, hi_ref, wi_ref, start_ref, g_ref,
                 i0_ref, i1_ref, i2_ref, i3_ref,
                 w0_ref, w1_ref, w2_ref, w3_ref, lg_ref):
    qb = q_ref[...]
    sx = jnp.dot(qb, wx_ref[...], preferred_element_type=jnp.float32) + bx_ref[...]
    sy = jnp.dot(qb, wy_ref[...], preferred_element_type=jnp.float32) + by_ref[...]
    lg_ref[...] = (
        jnp.dot(qb, wa_ref[...], preferred_element_type=jnp.float32) + ba_ref[...]
    )
    lsel = lsel_ref[...]
    rx = jnp.dot(rx_ref[...], lsel, preferred_element_type=jnp.float32,
                 precision=lax.Precision.HIGHEST)
    ry = jnp.dot(ry_ref[...], lsel, preferred_element_type=jnp.float32,
                 precision=lax.Precision.HIGHEST)
    hf = hi_ref[...].astype(jnp.float32)
    wf = wi_ref[...].astype(jnp.float32)
    lx = rx + sx * (1.0 / wf)
    ly = ry + sy * (1.0 / hf)
    wim = lx * wf - 0.5
    him = ly * hf - 0.5
    h0f = jnp.floor(him)
    w0f = jnp.floor(wim)
    lh = him - h0f
    lw = wim - w0f
    hh = 1.0 - lh
    hw = 1.0 - lw
    h0 = h0f.astype(jnp.int32)
    w0 = w0f.astype(jnp.int32)
    h1 = h0 + 1
    w1 = w0 + 1
    validf = ((him > -1.0) & (wim > -1.0) & (him < hf) & (wim < wf)).astype(
        jnp.float32)
    hi = hi_ref[...]
    wi = wi_ref[...]
    start = start_ref[...]
    glane = g_ref[...]
    boff = pl.program_id(0) * (_LEN_IN * _G)
    iouts = (i0_ref, i1_ref, i2_ref, i3_ref)
    wouts = (w0_ref, w1_ref, w2_ref, w3_ref)
    corners = ((h0, w0, hh * hw), (h0, w1, hh * lw),
               (h1, w0, lh * hw), (h1, w1, lh * lw))
    for (hc_, wc_, bw), i_ref, w_ref in zip(corners, iouts, wouts):
        m = ((hc_ >= 0) & (hc_ < hi) & (wc_ >= 0) & (wc_ < wi)).astype(
            jnp.float32)
        hcl = jnp.clip(hc_, 0, hi - 1)
        wcl = jnp.clip(wc_, 0, wi - 1)
        i_ref[...] = (start + hcl * wi + wcl) * _G + glane + boff
        w_ref[...] = bw * m * validf


def _prep(q2, wx, wy, wa, bx, by, ba, rx, ry):
    nlane = _G * _L * _K
    nb = _Q // _QB
    io = jax.ShapeDtypeStruct((_B * _Q, nlane), jnp.int32)
    wo = jax.ShapeDtypeStruct((_B * _Q, nlane), jnp.float32)
    blk = lambda i, b=None: (i, 0)
    return pl.pallas_call(
        _prep_kernel,
        grid=(_B, nb),
        in_specs=[
            pl.BlockSpec((_QB, _DM), lambda b, i: (b * nb + i, 0)),
            pl.BlockSpec((_DM, nlane), lambda b, i: (0, 0)),
            pl.BlockSpec((_DM, nlane), lambda b, i: (0, 0)),
            pl.BlockSpec((_DM, nlane), lambda b, i: (0, 0)),
            pl.BlockSpec((1, nlane), lambda b, i: (0, 0)),
            pl.BlockSpec((1, nlane), lambda b, i: (0, 0)),
            pl.BlockSpec((1, nlane), lambda b, i: (0, 0)),
            pl.BlockSpec((_QB, _L), lambda b, i: (b * nb + i, 0)),
            pl.BlockSpec((_QB, _L), lambda b, i: (b * nb + i, 0)),
            pl.BlockSpec((_L, nlane), lambda b, i: (0, 0)),
            pl.BlockSpec((1, nlane), lambda b, i: (0, 0)),
            pl.BlockSpec((1, nlane), lambda b, i: (0, 0)),
            pl.BlockSpec((1, nlane), lambda b, i: (0, 0)),
            pl.BlockSpec((1, nlane), lambda b, i: (0, 0)),
        ],
        out_specs=[pl.BlockSpec((_QB, nlane), lambda b, i: (b * nb + i, 0))] * 9,
        out_shape=[io, io, io, io, wo, wo, wo, wo, wo],
    )(q2, wx, wy, wa, bx, by, ba, rx, ry,
      jnp.asarray(_LSEL), jnp.asarray(_LANE_HI), jnp.asarray(_LANE_WI),
      jnp.asarray(_LANE_START), jnp.asarray(_LANE_G))


def _lane_bcast(x, idx):
    dn = lax.GatherDimensionNumbers(
        offset_dims=(), collapsed_slice_dims=(0,), start_index_map=(0,))
    return lax.gather(x, idx[:, None], dn, slice_sizes=(1,),
                      mode=lax.GatherScatterMode.PROMISE_IN_BOUNDS)


def _all_max(x):
    lane = lax.iota(jnp.int32, 16)
    for k in range(4):
        x = jnp.maximum(x, _lane_bcast(x, lane ^ (1 << k)))
    return x


def _all_sum(x):
    lane = lax.iota(jnp.int32, 16)
    for k in range(4):
        x = x + _lane_bcast(x, lane ^ (1 << k))
    return x


def _sc_compute(sw, rv, outv):
    for o in range(_NB):
        lgv = sw[4, o]
        mx = _all_max(lgv)
        e = jnp.exp(lgv - mx)
        rinv = 1.0 / _all_sum(e)
        ew = e * rinv
        wfin = [sw[c, o] * ew for c in range(4)]
        acc0 = jnp.zeros((16,), jnp.float32)
        acc1 = jnp.zeros((16,), jnp.float32)
        for c in range(4):
            for j in range(16):
                wb = _lane_bcast(wfin[c], jnp.full((16,), j, jnp.int32))
                row = o * 16 + j
                acc0 = acc0 + wb * rv[c, row, pl.ds(0, 16)]
                acc1 = acc1 + wb * rv[c, row, pl.ds(16, 16)]
        outv[o, pl.ds(0, 16)] = acc0
        outv[o, pl.ds(16, 16)] = acc1


def _sc_body(table, ipk, wpk, out,
             ie, io_, se, so, re_, ro, outve, outvo,
             sem_se, sem_so, sem_ge, sem_go, sem_oe, sem_oo):
    wid = lax.axis_index("s") * 2 + lax.axis_index("c")
    nsub = _CHUNK // _NB
    base = wid * nsub

    def pair(j2, carry):
        j = base + j2 * 2
        dse = [pltpu.async_copy(ipk.at[j], ie, sem_se),
               pltpu.async_copy(wpk.at[j], se, sem_se)]
        dso = [pltpu.async_copy(ipk.at[j + 1], io_, sem_so),
               pltpu.async_copy(wpk.at[j + 1], so, sem_so)]
        for d in dse:
            d.wait()
        dge = [pltpu.async_copy(table.at[ie.at[c]], re_.at[c], sem_ge)
               for c in range(4)]
        for d in dso:
            d.wait()
        dgo = [pltpu.async_copy(table.at[io_.at[c]], ro.at[c], sem_go)
               for c in range(4)]
        for d in dge:
            d.wait()
        _sc_compute(se, re_, outve)
        dwe = pltpu.async_copy(outve, out.at[pl.ds(j * _NB, _NB)], sem_oe)
        for d in dgo:
            d.wait()
        _sc_compute(so, ro, outvo)
        dwo = pltpu.async_copy(outvo, out.at[pl.ds((j + 1) * _NB, _NB)], sem_oo)
        dwe.wait()
        dwo.wait()
        return carry

    lax.fori_loop(0, nsub // 2, pair, 0)


@functools.partial(jax.jit)
def _sc_sample(table, ipk, wpk):
    mesh = plsc.VectorSubcoreMesh(core_axis_name="c", subcore_axis_name="s",
                                  num_cores=2, num_subcores=16)
    f = pl.kernel(
        _sc_body,
        out_type=jax.ShapeDtypeStruct((_N_OUT, _DH), jnp.float32),
        mesh=mesh,
        scratch_types=[
            pltpu.VMEM((4, _NROW), jnp.int32),
            pltpu.VMEM((4, _NROW), jnp.int32),
            pltpu.VMEM((5, _NB, 16), jnp.float32),
            pltpu.VMEM((5, _NB, 16), jnp.float32),
            pltpu.VMEM((4, _NROW, _DH), jnp.float32),
            pltpu.VMEM((4, _NROW, _DH), jnp.float32),
            pltpu.VMEM((_NB, _DH), jnp.float32),
            pltpu.VMEM((_NB, _DH), jnp.float32),
            pltpu.SemaphoreType.DMA,
            pltpu.SemaphoreType.DMA,
            pltpu.SemaphoreType.DMA,
            pltpu.SemaphoreType.DMA,
            pltpu.SemaphoreType.DMA,
            pltpu.SemaphoreType.DMA,
        ],
        compiler_params=pltpu.CompilerParams(use_tc_tiling_on_sc=False),
    )
    return f(table, ipk, wpk)


def kernel(query, reference_points, input_flatten, spatial_shapes,
           level_start_index, W_samp, b_samp, W_attn, b_attn, W_val, b_val,
           W_out, b_out):
    q2 = query.reshape(_B * _Q, _DM)
    ws = W_samp.reshape(_DM, _G, _L, _K, 2)
    wx = ws[..., 0].reshape(_DM, -1)
    wy = ws[..., 1].reshape(_DM, -1)
    bs = b_samp.reshape(_G, _L, _K, 2)
    bx = bs[..., 0].reshape(1, -1)
    by = bs[..., 1].reshape(1, -1)
    rx = reference_points[..., 0].reshape(_B * _Q, _L)
    ry = reference_points[..., 1].reshape(_B * _Q, _L)

    i0, i1, i2, i3, w0, w1, w2, w3, lgq = _prep(
        q2, wx, wy, W_attn, bx, by, b_attn.reshape(1, -1), rx, ry)

    value = _matmul_bias(input_flatten.reshape(_B * _LEN_IN, _DM), W_val, b_val)
    table = value.reshape(_B * _LEN_IN * _G, _DH)

    nsb = _N_OUT // _NB
    ipk = jnp.stack([a.reshape(nsb, _NB * 16) for a in (i0, i1, i2, i3)],
                    axis=1)
    wpk = jnp.stack([a.reshape(nsb, _NB, 16)
                     for a in (w0, w1, w2, w3, lgq)], axis=1)
    out_sc = _sc_sample(table, ipk, wpk)

    out = _matmul_bias(out_sc.reshape(_B * _Q, _DM), W_out, b_out)
    return out.reshape(_B, _Q, _DM)


# speedup vs baseline: 1735.3095x; 1.0581x over previous
import functools
import math

import jax
import jax.numpy as jnp
import numpy as np
from jax import lax
from jax.experimental import pallas as pl
from jax.experimental.pallas import tpu as pltpu
from jax.experimental.pallas import tpu_sc as plsc

_B, _Q, _DM = 2, 5440, 256
_G, _L, _K = 8, 4, 4
_DH = _DM // _G
_SPATIAL = np.array([[64, 64], [32, 32], [16, 16], [8, 8]], dtype=np.int64)
_LEVEL_START = np.array([0, 4096, 5120, 5376], dtype=np.int64)
_LEN_IN = 5440
_N_OUT = _B * _Q * _G
_QB = 544
_NTILES = 32
_CHUNK = _N_OUT // _NTILES
_NB = 8
_NROW = _NB * _L * _K

_lane = np.arange(_G * _L * _K)
_lane_l = (_lane // _K) % _L
_LANE_H = _SPATIAL[_lane_l, 0].astype(np.float32).reshape(1, -1)
_LANE_W = _SPATIAL[_lane_l, 1].astype(np.float32).reshape(1, -1)
_LANE_HI = _SPATIAL[_lane_l, 0].astype(np.int32).reshape(1, -1)
_LANE_WI = _SPATIAL[_lane_l, 1].astype(np.int32).reshape(1, -1)
_LANE_START = _LEVEL_START[_lane_l].astype(np.int32).reshape(1, -1)
_LANE_G = (_lane // (_L * _K)).astype(np.int32).reshape(1, -1)
_LSEL = np.zeros((_L, _G * _L * _K), np.float32)
_LSEL[_lane_l, _lane] = 1.0


def _matmul_bias_kernel(x_ref, w_ref, b_ref, o_ref):
    o_ref[...] = (
        jnp.dot(x_ref[...], w_ref[...], preferred_element_type=jnp.float32)
        + b_ref[...]
    )


def _matmul_bias(x, w, b, bm=2176):
    m, k = x.shape
    _, n = w.shape
    return pl.pallas_call(
        _matmul_bias_kernel,
        grid=(m // bm,),
        in_specs=[
            pl.BlockSpec((bm, k), lambda i: (i, 0)),
            pl.BlockSpec((k, n), lambda i: (0, 0)),
            pl.BlockSpec((1, n), lambda i: (0, 0)),
        ],
        out_specs=pl.BlockSpec((bm, n), lambda i: (i, 0)),
        out_shape=jax.ShapeDtypeStruct((m, n), jnp.float32),
    )(x, w, b.reshape(1, n))


def _prep_kernel(q_ref, wx_ref, wy_ref, wa_ref, bx_ref, by_ref, ba_ref,
                 rx_ref, ry_ref, lsel_ref, hi_ref, wi_ref, start_ref, g_ref,
                 i0_ref, i1_ref, i2_ref, i3_ref,
                 w0_ref, w1_ref, w2_ref, w3_ref, lg_ref):
    qb = q_ref[...]
    sx = jnp.dot(qb, wx_ref[...], preferred_element_type=jnp.float32) + bx_ref[...]
    sy = jnp.dot(qb, wy_ref[...], preferred_element_type=jnp.float32) + by_ref[...]
    lg_ref[...] = (
        jnp.dot(qb, wa_ref[...], preferred_element_type=jnp.float32) + ba_ref[...]
    )
    lsel = lsel_ref[...]
    rx = jnp.dot(rx_ref[...], lsel, preferred_element_type=jnp.float32,
                 precision=lax.Precision.HIGHEST)
    ry = jnp.dot(ry_ref[...], lsel, preferred_element_type=jnp.float32,
                 precision=lax.Precision.HIGHEST)
    hf = hi_ref[...].astype(jnp.float32)
    wf = wi_ref[...].astype(jnp.float32)
    lx = rx + sx * (1.0 / wf)
    ly = ry + sy * (1.0 / hf)
    wim = lx * wf - 0.5
    him = ly * hf - 0.5
    h0f = jnp.floor(him)
    w0f = jnp.floor(wim)
    lh = him - h0f
    lw = wim - w0f
    hh = 1.0 - lh
    hw = 1.0 - lw
    h0 = h0f.astype(jnp.int32)
    w0 = w0f.astype(jnp.int32)
    h1 = h0 + 1
    w1 = w0 + 1
    validf = ((him > -1.0) & (wim > -1.0) & (him < hf) & (wim < wf)).astype(
        jnp.float32)
    hi = hi_ref[...]
    wi = wi_ref[...]
    start = start_ref[...]
    glane = g_ref[...]
    boff = pl.program_id(0) * (_LEN_IN * _G)
    iouts = (i0_ref, i1_ref, i2_ref, i3_ref)
    wouts = (w0_ref, w1_ref, w2_ref, w3_ref)
    corners = ((h0, w0, hh * hw), (h0, w1, hh * lw),
               (h1, w0, lh * hw), (h1, w1, lh * lw))
    for (hc_, wc_, bw), i_ref, w_ref in zip(corners, iouts, wouts):
        m = ((hc_ >= 0) & (hc_ < hi) & (wc_ >= 0) & (wc_ < wi)).astype(
            jnp.float32)
        hcl = jnp.clip(hc_, 0, hi - 1)
        wcl = jnp.clip(wc_, 0, wi - 1)
        i_ref[...] = (start + hcl * wi + wcl) * _G + glane + boff
        w_ref[...] = bw * m * validf


def _prep(q2, wx, wy, wa, bx, by, ba, rx, ry):
    nlane = _G * _L * _K
    nb = _Q // _QB
    io = jax.ShapeDtypeStruct((_B * _Q, nlane), jnp.int32)
    wo = jax.ShapeDtypeStruct((_B * _Q, nlane), jnp.float32)
    blk = lambda i, b=None: (i, 0)
    return pl.pallas_call(
        _prep_kernel,
        grid=(_B, nb),
        in_specs=[
            pl.BlockSpec((_QB, _DM), lambda b, i: (b * nb + i, 0)),
            pl.BlockSpec((_DM, nlane), lambda b, i: (0, 0)),
            pl.BlockSpec((_DM, nlane), lambda b, i: (0, 0)),
            pl.BlockSpec((_DM, nlane), lambda b, i: (0, 0)),
            pl.BlockSpec((1, nlane), lambda b, i: (0, 0)),
            pl.BlockSpec((1, nlane), lambda b, i: (0, 0)),
            pl.BlockSpec((1, nlane), lambda b, i: (0, 0)),
            pl.BlockSpec((_QB, _L), lambda b, i: (b * nb + i, 0)),
            pl.BlockSpec((_QB, _L), lambda b, i: (b * nb + i, 0)),
            pl.BlockSpec((_L, nlane), lambda b, i: (0, 0)),
            pl.BlockSpec((1, nlane), lambda b, i: (0, 0)),
            pl.BlockSpec((1, nlane), lambda b, i: (0, 0)),
            pl.BlockSpec((1, nlane), lambda b, i: (0, 0)),
            pl.BlockSpec((1, nlane), lambda b, i: (0, 0)),
        ],
        out_specs=[pl.BlockSpec((_QB, nlane), lambda b, i: (b * nb + i, 0))] * 9,
        out_shape=[io, io, io, io, wo, wo, wo, wo, wo],
    )(q2, wx, wy, wa, bx, by, ba, rx, ry,
      jnp.asarray(_LSEL), jnp.asarray(_LANE_HI), jnp.asarray(_LANE_WI),
      jnp.asarray(_LANE_START), jnp.asarray(_LANE_G))


def _lane_bcast(x, idx):
    dn = lax.GatherDimensionNumbers(
        offset_dims=(), collapsed_slice_dims=(0,), start_index_map=(0,))
    return lax.gather(x, idx[:, None], dn, slice_sizes=(1,),
                      mode=lax.GatherScatterMode.PROMISE_IN_BOUNDS)


def _all_max(x):
    lane = lax.iota(jnp.int32, 16)
    for k in range(4):
        x = jnp.maximum(x, _lane_bcast(x, lane ^ (1 << k)))
    return x


def _all_sum(x):
    lane = lax.iota(jnp.int32, 16)
    for k in range(4):
        x = x + _lane_bcast(x, lane ^ (1 << k))
    return x


def _sc_compute(sw, rv, outv):
    for o in range(_NB):
        lgv = sw[4, o]
        mx = _all_max(lgv)
        e = jnp.exp(lgv - mx)
        rinv = 1.0 / _all_sum(e)
        ew = e * rinv
        wfin = [sw[c, o] * ew for c in range(4)]
        acc0 = jnp.zeros((16,), jnp.float32)
        acc1 = jnp.zeros((16,), jnp.float32)
        for c in range(4):
            for j in range(16):
                wb = _lane_bcast(wfin[c], jnp.full((16,), j, jnp.int32))
                row = o * 16 + j
                acc0 = acc0 + wb * rv[c, row, pl.ds(0, 16)]
                acc1 = acc1 + wb * rv[c, row, pl.ds(16, 16)]
        outv[o, pl.ds(0, 16)] = acc0
        outv[o, pl.ds(16, 16)] = acc1


def _sc_body(table, ipk, wpk, out, *bufs):
    I = bufs[0:2]
    S = bufs[2:4]
    R = bufs[4:6]
    OV = bufs[6:8]
    SI = bufs[8:10]
    SW = bufs[10:12]
    SG = bufs[12:14]
    SO = bufs[14:16]
    E, O = 0, 1
    wid = lax.axis_index("s") * 2 + lax.axis_index("c")
    nsub = _CHUNK // _NB
    base = wid * nsub
    last = base + nsub - 1

    def fire_si(b, k):
        pltpu.async_copy(ipk.at[jnp.minimum(b, last)], I[k], SI[k])

    def fire_sw(b, k):
        pltpu.async_copy(wpk.at[jnp.minimum(b, last)], S[k], SW[k])

    def wait_si(k):
        pltpu.make_async_copy(ipk.at[base], I[k], SI[k]).wait()

    def wait_sw(k):
        pltpu.make_async_copy(wpk.at[base], S[k], SW[k]).wait()

    def fire_g(k):
        for c in range(4):
            pltpu.async_copy(table.at[I[k].at[c]], R[k].at[c], SG[k])

    def wait_g(k):
        for c in range(4):
            pltpu.make_async_copy(table.at[I[k].at[c]], R[k].at[c],
                                  SG[k]).wait()

    def wait_w(k):
        pltpu.make_async_copy(OV[k], out.at[pl.ds(base * _NB, _NB)],
                              SO[k]).wait()

    fire_si(base + 0, E)
    fire_si(base + 1, O)
    fire_sw(base + 0, E)
    fire_sw(base + 1, O)
    for k in (E, O):
        pltpu.async_copy(out.at[pl.ds(base * _NB, _NB)], OV[k], SO[k])
    wait_si(E)
    fire_g(E)

    def body(i, carry):
        b = base + i * 2
        wait_si(O)
        fire_g(O)
        wait_g(E)
        fire_si(b + 2, E)
        wait_sw(E)
        wait_w(E)
        _sc_compute(S[E], R[E], OV[E])
        pltpu.async_copy(OV[E], out.at[pl.ds(b * _NB, _NB)], SO[E])
        fire_sw(b + 2, E)
        wait_g(O)
        fire_si(b + 3, O)
        wait_sw(O)
        wait_w(O)
        _sc_compute(S[O], R[O], OV[O])
        pltpu.async_copy(OV[O], out.at[pl.ds((b + 1) * _NB, _NB)], SO[O])
        fire_sw(b + 3, O)
        wait_si(E)
        fire_g(E)
        return carry

    lax.fori_loop(0, nsub // 2, body, 0)
    wait_g(E)
    wait_si(O)
    wait_sw(E)
    wait_sw(O)
    wait_w(E)
    wait_w(O)


@functools.partial(jax.jit)
def _sc_sample(table, ipk, wpk):
    mesh = plsc.VectorSubcoreMesh(core_axis_name="c", subcore_axis_name="s",
                                  num_cores=2, num_subcores=16)
    f = pl.kernel(
        _sc_body,
        out_type=jax.ShapeDtypeStruct((_N_OUT, _DH), jnp.float32),
        mesh=mesh,
        scratch_types=(
            [pltpu.VMEM((4, _NROW), jnp.int32)] * 2
            + [pltpu.VMEM((5, _NB, 16), jnp.float32)] * 2
            + [pltpu.VMEM((4, _NROW, _DH), jnp.float32)] * 2
            + [pltpu.VMEM((_NB, _DH), jnp.float32)] * 2
            + [pltpu.SemaphoreType.DMA] * 8
        ),
        compiler_params=pltpu.CompilerParams(use_tc_tiling_on_sc=False),
    )
    return f(table, ipk, wpk)


def kernel(query, reference_points, input_flatten, spatial_shapes,
           level_start_index, W_samp, b_samp, W_attn, b_attn, W_val, b_val,
           W_out, b_out):
    q2 = query.reshape(_B * _Q, _DM)
    ws = W_samp.reshape(_DM, _G, _L, _K, 2)
    wx = ws[..., 0].reshape(_DM, -1)
    wy = ws[..., 1].reshape(_DM, -1)
    bs = b_samp.reshape(_G, _L, _K, 2)
    bx = bs[..., 0].reshape(1, -1)
    by = bs[..., 1].reshape(1, -1)
    rx = reference_points[..., 0].reshape(_B * _Q, _L)
    ry = reference_points[..., 1].reshape(_B * _Q, _L)

    i0, i1, i2, i3, w0, w1, w2, w3, lgq = _prep(
        q2, wx, wy, W_attn, bx, by, b_attn.reshape(1, -1), rx, ry)

    value = _matmul_bias(input_flatten.reshape(_B * _LEN_IN, _DM), W_val, b_val)
    table = value.reshape(_B * _LEN_IN * _G, _DH)

    nsb = _N_OUT // _NB
    ipk = jnp.stack([a.reshape(nsb, _NB * 16) for a in (i0, i1, i2, i3)],
                    axis=1)
    wpk = jnp.stack([a.reshape(nsb, _NB, 16)
                     for a in (w0, w1, w2, w3, lgq)], axis=1)
    out_sc = _sc_sample(table, ipk, wpk)

    out = _matmul_bias(out_sc.reshape(_B * _Q, _DM), W_out, b_out)
    return out.reshape(_B, _Q, _DM)
